# baseline (speedup 1.0000x reference)
import functools
import math

import jax
import jax.numpy as jnp
from jax import lax
from jax.experimental import pallas as pl
from jax.experimental.pallas import tpu as pltpu
from jax.experimental.pallas import tpu_sc as plsc

_VOCAB = 100000
_D = 128
_SCALE = math.sqrt(128.0)

_NC = 2
_NS = 16
_NW = _NC * _NS

_B = 4096 * 200
_C = 128
_CHUNKS_PER_W = _B // (_NW * _C)
_NBUF = 4
_WPR = _D // 2
_HI_MASK = -65536


def _prep_body(t_ref, o_ref):
    bf = (t_ref[...] * _SCALE).astype(jnp.bfloat16)
    for j in range(_D // 32):
        lo = lax.bitcast_convert_type(
            bf[:, 32 * j:32 * j + 16], jnp.uint16).astype(jnp.uint32)
        hi = lax.bitcast_convert_type(
            bf[:, 32 * j + 16:32 * j + 32], jnp.uint16).astype(jnp.uint32)
        o_ref[:, 16 * j:16 * j + 16] = lax.bitcast_convert_type(
            lo | (hi << 16), jnp.int32)


def _prep_table(table):
    rows_blk = 1000
    return pl.pallas_call(
        _prep_body,
        grid=(_VOCAB // rows_blk,),
        in_specs=[pl.BlockSpec((rows_blk, _D), lambda i: (i, 0))],
        out_specs=pl.BlockSpec((rows_blk, _WPR), lambda i: (i, 0)),
        out_shape=jax.ShapeDtypeStruct((_VOCAB, _WPR), jnp.int32),
    )(table)


def _gather_body(table_hbm, idx_hbm, out_hbm, idx_v,
                 i0, i1, i2, i3, o0, o1, o2, o3,
                 g0, g1, g2, g3, s0, s1, s2, s3):
    wid = lax.axis_index("s") * _NC + lax.axis_index("c")
    row0 = wid * _CHUNKS_PER_W
    pltpu.sync_copy(idx_hbm.at[pl.ds(row0, _CHUNKS_PER_W)], idx_v)

    rin = (i0, i1, i2, i3)
    rout = (o0, o1, o2, o3)
    gsem = (g0, g1, g2, g3)
    ssem = (s0, s1, s2, s3)

    def start_gather(b, c):
        pltpu.make_async_copy(table_hbm.at[idx_v.at[c]], rin[b],
                              gsem[b]).start()

    def wait_gather(b):
        pltpu.make_async_copy(table_hbm.at[pl.ds(0, _C)], rin[b],
                              gsem[b]).wait()

    def start_store(b, c):
        pltpu.make_async_copy(rout[b], out_hbm.at[pl.ds((row0 + c) * _C, _C)],
                              ssem[b]).start()

    def wait_store(b):
        pltpu.make_async_copy(rout[b], out_hbm.at[pl.ds(0, _C)],
                              ssem[b]).wait()

    def expand(b):
        src = rin[b]
        dst = rout[b]

        @plsc.parallel_loop(0, _C, step=1, unroll=2)
        def _(i):
            for j in range(_D // 32):
                w = src[i, pl.ds(16 * j, 16)]
                dst[i, pl.ds(32 * j, 16)] = lax.bitcast_convert_type(
                    w << 16, jnp.float32)
                dst[i, pl.ds(32 * j + 16, 16)] = lax.bitcast_convert_type(
                    w & _HI_MASK, jnp.float32)

    for b in range(_NBUF):
        start_gather(b, b)
    for s in range(_NBUF):
        wait_gather(s)
        expand(s)
        start_store(s, s)
        start_gather(s, s + _NBUF)

    def body(g, carry):
        for b in range(_NBUF):
            c = g * _NBUF + b
            wait_gather(b)
            wait_store(b)
            expand(b)
            start_store(b, c)
            start_gather(b, c + _NBUF)
        return carry

    lax.fori_loop(1, _CHUNKS_PER_W // _NBUF - 1, body, 0)

    n = _CHUNKS_PER_W
    for b in range(_NBUF):
        wait_gather(b)
        wait_store(b)
        expand(b)
        start_store(b, n - _NBUF + b)
    for b in range(_NBUF):
        wait_store(b)


def _gather(table_i32, idx2d):
    f = functools.partial(
        pl.kernel,
        mesh=plsc.VectorSubcoreMesh(core_axis_name="c", subcore_axis_name="s"),
        compiler_params=pltpu.CompilerParams(use_tc_tiling_on_sc=False),
        out_type=jax.ShapeDtypeStruct((_B, _D), jnp.float32),
        scratch_types=(
            [pltpu.VMEM((_CHUNKS_PER_W, _C), jnp.int32)]
            + [pltpu.VMEM((_C, _WPR), jnp.int32)] * _NBUF
            + [pltpu.VMEM((_C, _D), jnp.float32)] * _NBUF
            + [pltpu.SemaphoreType.DMA] * (2 * _NBUF)
        ),
    )(_gather_body)
    return f(table_i32, idx2d)


def kernel(x, table):
    idx2d = x.reshape(_B // _C, _C).astype(jnp.int32)
    tbl = _prep_table(table)
    out = _gather(tbl, idx2d)
    return out.reshape(4096, 200, _D)


# speedup vs baseline: 1.1006x; 1.1006x over previous
import functools
import math

import jax
import jax.numpy as jnp
from jax import lax
from jax.experimental import pallas as pl
from jax.experimental.pallas import tpu as pltpu
from jax.experimental.pallas import tpu_sc as plsc

_VOCAB = 100000
_D = 128
_SCALE = math.sqrt(128.0)

_NC = 2
_NS = 16
_NW = _NC * _NS

_B = 4096 * 200
_C = 128
_CHUNKS_PER_W = _B // (_NW * _C)
_NBUF = 4
_WPR = _D // 2
_HI_MASK = -65536


def _gather_body(table_hbm, idx_hbm, out_hbm, idx_v,
                 i0, i1, i2, i3, o0, o1, o2, o3,
                 g0, g1, g2, g3, s0, s1, s2, s3):
    wid = lax.axis_index("s") * _NC + lax.axis_index("c")
    row0 = wid * _CHUNKS_PER_W
    pltpu.sync_copy(idx_hbm.at[pl.ds(row0, _CHUNKS_PER_W)], idx_v)

    rin = (i0, i1, i2, i3)
    rout = (o0, o1, o2, o3)
    gsem = (g0, g1, g2, g3)
    ssem = (s0, s1, s2, s3)

    def start_gather(b, c):
        pltpu.make_async_copy(table_hbm.at[idx_v.at[c]], rin[b],
                              gsem[b]).start()

    def wait_gather(b):
        pltpu.make_async_copy(table_hbm.at[pl.ds(0, _C)], rin[b],
                              gsem[b]).wait()

    def start_store(b, c):
        pltpu.make_async_copy(rout[b], out_hbm.at[pl.ds((row0 + c) * _C, _C)],
                              ssem[b]).start()

    def wait_store(b):
        pltpu.make_async_copy(rout[b], out_hbm.at[pl.ds(0, _C)],
                              ssem[b]).wait()

    def expand(b):
        src = rin[b]
        dst = rout[b]

        @plsc.parallel_loop(0, _C, step=1, unroll=2)
        def _(i):
            for j in range(_WPR // 16):
                w = src[i, pl.ds(16 * j, 16)]
                lo = lax.bitcast_convert_type(w << 16, jnp.float32)
                hi = lax.bitcast_convert_type(w & _HI_MASK, jnp.float32)
                dst[i, pl.ds(16 * j, 16)] = lo * _SCALE
                dst[i, pl.ds(_WPR + 16 * j, 16)] = hi * _SCALE

    for b in range(_NBUF):
        start_gather(b, b)
    for s in range(_NBUF):
        wait_gather(s)
        expand(s)
        start_store(s, s)
        start_gather(s, s + _NBUF)

    def body(g, carry):
        for b in range(_NBUF):
            c = g * _NBUF + b
            wait_gather(b)
            wait_store(b)
            expand(b)
            start_store(b, c)
            start_gather(b, c + _NBUF)
        return carry

    lax.fori_loop(1, _CHUNKS_PER_W // _NBUF - 1, body, 0)

    n = _CHUNKS_PER_W
    for b in range(_NBUF):
        wait_gather(b)
        wait_store(b)
        expand(b)
        start_store(b, n - _NBUF + b)
    for b in range(_NBUF):
        wait_store(b)


def _gather(table_i32, idx2d):
    f = functools.partial(
        pl.kernel,
        mesh=plsc.VectorSubcoreMesh(core_axis_name="c", subcore_axis_name="s"),
        compiler_params=pltpu.CompilerParams(use_tc_tiling_on_sc=False),
        out_type=jax.ShapeDtypeStruct((_B, _D), jnp.float32),
        scratch_types=(
            [pltpu.VMEM((_CHUNKS_PER_W, _C), jnp.int32)]
            + [pltpu.VMEM((_C, _WPR), jnp.int32)] * _NBUF
            + [pltpu.VMEM((_C, _D), jnp.float32)] * _NBUF
            + [pltpu.SemaphoreType.DMA] * (2 * _NBUF)
        ),
    )(_gather_body)
    return f(table_i32, idx2d)


def _pack_table(table):
    bf = table.astype(jnp.bfloat16)
    lo = lax.bitcast_convert_type(bf[:, :_WPR], jnp.uint16).astype(jnp.uint32)
    hi = lax.bitcast_convert_type(bf[:, _WPR:], jnp.uint16).astype(jnp.uint32)
    return lax.bitcast_convert_type(lo | (hi << 16), jnp.int32)


def kernel(x, table):
    idx2d = x.reshape(_B // _C, _C).astype(jnp.int32)
    out = _gather(_pack_table(table), idx2d)
    return out.reshape(4096, 200, _D)


# speedup vs baseline: 1.2344x; 1.1216x over previous
import functools
import math

import jax
import jax.numpy as jnp
from jax import lax
from jax.experimental import pallas as pl
from jax.experimental.pallas import tpu as pltpu
from jax.experimental.pallas import tpu_sc as plsc

_VOCAB = 100000
_D = 128
_SCALE = math.sqrt(128.0)

_NC = 2
_NS = 16
_NW = _NC * _NS

_B = 4096 * 200
_C = 128
_CHUNKS_PER_W = _B // (_NW * _C)
_NBUF = 6
_LOOK = 3


def _gather_body(table_hbm, idx_hbm, out_hbm, idx_v,
                 r0, r1, r2, r3, r4, r5,
                 g0, g1, g2, g3, g4, g5, s0, s1, s2, s3, s4, s5):
    wid = lax.axis_index("s") * _NC + lax.axis_index("c")
    row0 = wid * _CHUNKS_PER_W
    pltpu.sync_copy(idx_hbm.at[pl.ds(row0, _CHUNKS_PER_W)], idx_v)

    rows = (r0, r1, r2, r3, r4, r5)
    gsem = (g0, g1, g2, g3, g4, g5)
    ssem = (s0, s1, s2, s3, s4, s5)

    def start_gather(b, c):
        pltpu.make_async_copy(table_hbm.at[idx_v.at[c]], rows[b],
                              gsem[b]).start()

    def wait_gather(b):
        pltpu.make_async_copy(table_hbm.at[pl.ds(0, _C)], rows[b],
                              gsem[b]).wait()

    def start_store(b, c):
        pltpu.make_async_copy(rows[b], out_hbm.at[pl.ds((row0 + c) * _C, _C)],
                              ssem[b]).start()

    def wait_store(b):
        pltpu.make_async_copy(rows[b], out_hbm.at[pl.ds(0, _C)],
                              ssem[b]).wait()

    def scale(b):
        r = rows[b]

        @plsc.parallel_loop(0, _C, step=1, unroll=4)
        def _(i):
            for j in range(_D // 16):
                sl = (i, pl.ds(j * 16, 16))
                r[sl] = r[sl] * _SCALE

    def slot(b, c, wait_prev_store, next_c):
        wait_gather(b)
        if next_c is not None:
            b3 = (b + _LOOK) % _NBUF
            if wait_prev_store:
                wait_store(b3)
            start_gather(b3, next_c)
        scale(b)
        start_store(b, c)

    for c in range(_LOOK):
        start_gather(c, c)
    for s in range(_NBUF):
        slot(s, s, wait_prev_store=(s >= _LOOK), next_c=s + _LOOK)

    def body(g, carry):
        for b in range(_NBUF):
            c = g * _NBUF + b
            slot(b, c, wait_prev_store=True, next_c=c + _LOOK)
        return carry

    lax.fori_loop(1, 32, body, 0)

    n = _CHUNKS_PER_W
    for s in range(192, n):
        nc = s + _LOOK
        slot(s % _NBUF, s, wait_prev_store=True,
             next_c=nc if nc < n else None)

    for s in range(n - _NBUF, n):
        wait_store(s % _NBUF)


def _gather(table, idx2d):
    f = functools.partial(
        pl.kernel,
        mesh=plsc.VectorSubcoreMesh(core_axis_name="c", subcore_axis_name="s"),
        out_type=jax.ShapeDtypeStruct((_B, _D), jnp.float32),
        scratch_types=(
            [pltpu.VMEM((_CHUNKS_PER_W, _C), jnp.int32)]
            + [pltpu.VMEM((_C, _D), jnp.float32)] * _NBUF
            + [pltpu.SemaphoreType.DMA] * (2 * _NBUF)
        ),
    )(_gather_body)
    return f(table, idx2d)


def kernel(x, table):
    idx2d = x.reshape(_B // _C, _C).astype(jnp.int32)
    out = _gather(table, idx2d)
    return out.reshape(4096, 200, _D)
